# Initial kernel scaffold; baseline (speedup 1.0000x reference)
import jax
import jax.numpy as jnp
from jax.experimental import pallas as pl


def kernel(x, edge_index, W1, b1, g1, be1, W2, b2, g2, be2):
    raise NotImplementedError("write your pallas kernel here")


# speedup vs baseline: 13.2018x; 13.2018x over previous
import functools

import jax
import jax.numpy as jnp
from jax import lax
from jax.experimental import pallas as pl
from jax.experimental.pallas import tpu as pltpu
from jax.experimental.pallas import tpu_sc as plsc

NN = 10000
FD = 128
NE = 320000
NC = 2
NS = 16
NW = NC * NS
NPAD = 10240
RPT = NPAD // NS
EPT = 10240
EPAD = EPT * NW
CH = 128
NCHUNK = EPT // CH
RCH = NCHUNK and RPT // CH
EPS = 1e-5


def _mesh():
    return plsc.VectorSubcoreMesh(core_axis_name="c", subcore_axis_name="s")


def _deg_body(dst_hbm, out_hbm, dst_v, ones_v, stg1, stg2, acc):
    c = lax.axis_index("c")
    s = lax.axis_index("s")
    w = c * NS + s
    for t in range(CH // 16):
        ones_v[pl.ds(16 * t, 16)] = jnp.ones((16,), jnp.float32)
    for t in range(RPT // 16):
        stg1[pl.ds(16 * t, 16)] = jnp.zeros((16,), jnp.float32)
    pltpu.sync_copy(stg1, acc.at[pl.ds(s * RPT, RPT)])
    plsc.subcore_barrier()

    def body(i, carry):
        ebase = w * EPT + i * CH
        pltpu.sync_copy(dst_hbm.at[pl.ds(ebase, CH)], dst_v)
        pltpu.sync_copy(ones_v, acc.at[dst_v], add=True)
        return carry

    lax.fori_loop(0, NCHUNK, body, 0)
    plsc.subcore_barrier()
    pltpu.sync_copy(acc.at[pl.ds(s * RPT, RPT)], stg1)
    for t in range(RPT // 16):
        stg2[t // 8, pl.ds((t % 8) * 16, 16)] = stg1[pl.ds(16 * t, 16)]
    pltpu.sync_copy(stg2, out_hbm.at[c, s])


def _deg_call(dstp):
    k = pl.kernel(
        _deg_body,
        out_type=jax.ShapeDtypeStruct((NC, NS, RPT // FD, FD), jnp.float32),
        mesh=_mesh(),
        scratch_types=[
            pltpu.VMEM((CH,), jnp.int32),
            pltpu.VMEM((CH,), jnp.float32),
            pltpu.VMEM((RPT,), jnp.float32),
            pltpu.VMEM((RPT // FD, FD), jnp.float32),
            pltpu.VMEM_SHARED((NPAD,), jnp.float32),
        ],
    )
    return k(dstp)


def _agg_body(hp_hbm, src_hbm, dst_hbm, out_hbm,
              src_v, dst_v, rows_v, stage_v, sem, acc):
    c = lax.axis_index("c")
    s = lax.axis_index("s")
    w = c * NS + s

    def initb(j, carry):
        rb = s * RPT + j * CH
        pltpu.sync_copy(hp_hbm.at[pl.ds(rb, CH), :], stage_v)
        pltpu.sync_copy(stage_v, acc.at[pl.ds(rb, CH), :])
        return carry

    lax.fori_loop(0, RCH, initb, 0)
    plsc.subcore_barrier()

    def body(i, carry):
        ebase = w * EPT + i * CH
        pltpu.sync_copy(src_hbm.at[pl.ds(ebase, CH)], src_v)
        pltpu.sync_copy(dst_hbm.at[pl.ds(ebase, CH)], dst_v)
        pltpu.async_copy(hp_hbm.at[src_v], rows_v, sem).wait()
        pltpu.sync_copy(rows_v, acc.at[dst_v], add=True)
        return carry

    lax.fori_loop(0, NCHUNK, body, 0)
    plsc.subcore_barrier()

    def wb(j, carry):
        rb = s * RPT + j * CH
        pltpu.sync_copy(acc.at[pl.ds(rb, CH), :], stage_v)
        pltpu.sync_copy(stage_v, out_hbm.at[c, pl.ds(rb, CH), :])
        return carry

    lax.fori_loop(0, RCH, wb, 0)


def _agg_call(hp, srcp, dstp):
    k = pl.kernel(
        _agg_body,
        out_type=jax.ShapeDtypeStruct((NC, NPAD, FD), jnp.float32),
        mesh=_mesh(),
        scratch_types=[
            pltpu.VMEM((CH,), jnp.int32),
            pltpu.VMEM((CH,), jnp.int32),
            pltpu.VMEM((CH, FD), jnp.float32),
            pltpu.VMEM((CH, FD), jnp.float32),
            pltpu.SemaphoreType.DMA,
            pltpu.VMEM_SHARED((NPAD, FD), jnp.float32),
        ],
    )
    return k(hp, srcp, dstp)


def _scale_body(x_ref, w_ref, d0_ref, d1_ref, hp_ref, dm_ref):
    i = pl.program_id(0)
    d0 = d0_ref[pl.ds(i, 1), :]
    d1 = d1_ref[pl.ds(i, 1), :]
    dis_row = lax.rsqrt(d0 + d1 + 1.0)
    ones_row = jnp.ones((1, FD), jnp.float32)
    dm = lax.dot_general(dis_row, ones_row, (((0,), (0,)), ((), ())),
                         preferred_element_type=jnp.float32)
    h = jnp.dot(x_ref[...], w_ref[...], preferred_element_type=jnp.float32)
    dm_ref[...] = dm
    hp_ref[...] = dm * h


def _scale_call(xp, W1, deg0, deg1):
    grid = NPAD // FD
    return pl.pallas_call(
        _scale_body,
        grid=(grid,),
        in_specs=[
            pl.BlockSpec((FD, FD), lambda i: (i, 0)),
            pl.BlockSpec((FD, FD), lambda i: (0, 0)),
            pl.BlockSpec((NPAD // FD, FD), lambda i: (0, 0)),
            pl.BlockSpec((NPAD // FD, FD), lambda i: (0, 0)),
        ],
        out_specs=[
            pl.BlockSpec((FD, FD), lambda i: (i, 0)),
            pl.BlockSpec((FD, FD), lambda i: (i, 0)),
        ],
        out_shape=[
            jax.ShapeDtypeStruct((NPAD, FD), jnp.float32),
            jax.ShapeDtypeStruct((NPAD, FD), jnp.float32),
        ],
    )(xp, W1, deg0, deg1)


def _ln_relu(z, g_ref, be_ref):
    mu = jnp.mean(z, axis=-1, keepdims=True)
    zc = z - mu
    var = jnp.mean(zc * zc, axis=-1, keepdims=True)
    zn = zc * lax.rsqrt(var + EPS) * g_ref[...] + be_ref[...]
    return jnp.maximum(zn, 0.0)


def _mid_body(a0_ref, a1_ref, hp_ref, dm_ref, b_ref, g_ref, be_ref, w2_ref,
              out_ref):
    dm = dm_ref[...]
    z = dm * (a0_ref[...] + a1_ref[...] - hp_ref[...]) + b_ref[...]
    r = _ln_relu(z, g_ref, be_ref)
    h2 = jnp.dot(r, w2_ref[...], preferred_element_type=jnp.float32)
    out_ref[...] = dm * h2


def _mid_call(a0, a1, hp, dm, b1, g1, be1, W2):
    grid = NPAD // FD
    blk = lambda i: (i, 0)
    vec = lambda i: (0, 0)
    return pl.pallas_call(
        _mid_body,
        grid=(grid,),
        in_specs=[
            pl.BlockSpec((FD, FD), blk),
            pl.BlockSpec((FD, FD), blk),
            pl.BlockSpec((FD, FD), blk),
            pl.BlockSpec((FD, FD), blk),
            pl.BlockSpec((1, FD), vec),
            pl.BlockSpec((1, FD), vec),
            pl.BlockSpec((1, FD), vec),
            pl.BlockSpec((FD, FD), vec),
        ],
        out_specs=pl.BlockSpec((FD, FD), blk),
        out_shape=jax.ShapeDtypeStruct((NPAD, FD), jnp.float32),
    )(a0, a1, hp, dm, b1, g1, be1, W2)


def _final_body(a0_ref, a1_ref, hp_ref, dm_ref, b_ref, g_ref, be_ref,
                out_ref):
    i = pl.program_id(0)
    z = dm_ref[...] * (a0_ref[...] + a1_ref[...] - hp_ref[...]) + b_ref[...]
    r = _ln_relu(z, g_ref, be_ref)
    rowid = lax.broadcasted_iota(jnp.int32, (FD, FD), 0) + i * FD
    r = jnp.where(rowid < NN, r, 0.0)
    part = jnp.dot(jnp.ones((1, FD), jnp.float32), r,
                   preferred_element_type=jnp.float32)

    @pl.when(i == 0)
    def _():
        out_ref[...] = jnp.zeros((1, FD), jnp.float32)

    out_ref[...] += part

    @pl.when(i == NPAD // FD - 1)
    def _():
        out_ref[...] = out_ref[...] * (1.0 / NN)


def _final_call(a0, a1, hp, dm, b2, g2, be2):
    grid = NPAD // FD
    blk = lambda i: (i, 0)
    vec = lambda i: (0, 0)
    return pl.pallas_call(
        _final_body,
        grid=(grid,),
        in_specs=[
            pl.BlockSpec((FD, FD), blk),
            pl.BlockSpec((FD, FD), blk),
            pl.BlockSpec((FD, FD), blk),
            pl.BlockSpec((FD, FD), blk),
            pl.BlockSpec((1, FD), vec),
            pl.BlockSpec((1, FD), vec),
            pl.BlockSpec((1, FD), vec),
        ],
        out_specs=pl.BlockSpec((1, FD), vec),
        out_shape=jax.ShapeDtypeStruct((1, FD), jnp.float32),
    )(a0, a1, hp, dm, b2, g2, be2)


def kernel(x, edge_index, W1, b1, g1, be1, W2, b2, g2, be2):
    npad_e = EPAD - NE
    pad_src = (jnp.arange(npad_e, dtype=jnp.int32) * 13) % NN
    pad_dst = NN + (jnp.arange(npad_e, dtype=jnp.int32) % (NPAD - NN))
    srcp = jnp.concatenate([edge_index[0].astype(jnp.int32), pad_src])
    dstp = jnp.concatenate([edge_index[1].astype(jnp.int32), pad_dst])
    xp = jnp.pad(x, ((0, NPAD - NN), (0, 0)))

    degs = _deg_call(dstp).reshape(NC, NPAD // FD, FD)
    hp1, dm = _scale_call(xp, W1, degs[0], degs[1])
    acc1 = _agg_call(hp1, srcp, dstp)
    hp2 = _mid_call(acc1[0], acc1[1], hp1, dm,
                    b1.reshape(1, FD), g1.reshape(1, FD), be1.reshape(1, FD),
                    W2)
    acc2 = _agg_call(hp2, srcp, dstp)
    return _final_call(acc2[0], acc2[1], hp2, dm,
                       b2.reshape(1, FD), g2.reshape(1, FD),
                       be2.reshape(1, FD))


# speedup vs baseline: 19.0983x; 1.4466x over previous
import functools

import jax
import jax.numpy as jnp
from jax import lax
from jax.experimental import pallas as pl
from jax.experimental.pallas import tpu as pltpu
from jax.experimental.pallas import tpu_sc as plsc

NN = 10000
FD = 128
NE = 320000
NC = 2
NS = 16
NW = NC * NS
NPAD = 10240
RPT = NPAD // NS
EPT = 10240
EPAD = EPT * NW
CH = 128
NCHUNK = EPT // CH
RCH = NCHUNK and RPT // CH
EPS = 1e-5


def _mesh():
    return plsc.VectorSubcoreMesh(core_axis_name="c", subcore_axis_name="s")


def _deg_body(dst_hbm, out_hbm, dsts_v, ones_v, stg1, stg2,
              sem0, sem1, sem2, sem3, acc):
    c = lax.axis_index("c")
    s = lax.axis_index("s")
    w = c * NS + s
    for t in range(CH // 16):
        ones_v[pl.ds(16 * t, 16)] = jnp.ones((16,), jnp.float32)
    for t in range(RPT // 16):
        stg1[pl.ds(16 * t, 16)] = jnp.zeros((16,), jnp.float32)
    pltpu.sync_copy(stg1, acc.at[pl.ds(s * RPT, RPT)])
    pltpu.sync_copy(dst_hbm.at[w], dsts_v)
    plsc.subcore_barrier()
    sems = [sem0, sem1, sem2, sem3]

    def body(i, carry):
        descs = []
        for b in range(4):
            descs.append(pltpu.async_copy(
                ones_v, acc.at[dsts_v.at[i * 4 + b]], sems[b], add=True))
        for d in descs:
            d.wait()
        return carry

    lax.fori_loop(0, NCHUNK // 4, body, 0)
    plsc.subcore_barrier()
    pltpu.sync_copy(acc.at[pl.ds(s * RPT, RPT)], stg1)
    for t in range(RPT // 16):
        stg2[t // 8, pl.ds((t % 8) * 16, 16)] = stg1[pl.ds(16 * t, 16)]
    pltpu.sync_copy(stg2, out_hbm.at[c, s])


def _deg_call(dstp):
    k = pl.kernel(
        _deg_body,
        out_type=jax.ShapeDtypeStruct((NC, NS, RPT // FD, FD), jnp.float32),
        mesh=_mesh(),
        scratch_types=[
            pltpu.VMEM((NCHUNK, CH), jnp.int32),
            pltpu.VMEM((CH,), jnp.float32),
            pltpu.VMEM((RPT,), jnp.float32),
            pltpu.VMEM((RPT // FD, FD), jnp.float32),
            pltpu.SemaphoreType.DMA,
            pltpu.SemaphoreType.DMA,
            pltpu.SemaphoreType.DMA,
            pltpu.SemaphoreType.DMA,
            pltpu.VMEM_SHARED((NPAD,), jnp.float32),
        ],
    )
    return k(dstp)


HNC = NCHUNK // 2


def _agg_body(hp_hbm, src_hbm, dst_hbm, out_hbm,
              srcs_v, dsts_v, ra, rb_, gsa, gsb, ssa, ssb, acc):
    c = lax.axis_index("c")
    s = lax.axis_index("s")
    w = c * NS + s

    def initb(j, carry):
        rbase = s * RPT + j * CH
        pltpu.sync_copy(hp_hbm.at[pl.ds(rbase, CH), :], ra)
        pltpu.sync_copy(ra, acc.at[pl.ds(rbase, CH), :])
        return carry

    lax.fori_loop(0, RCH, initb, 0)
    plsc.subcore_barrier()

    for h in range(2):
        pltpu.sync_copy(src_hbm.at[w, pl.ds(h * HNC, HNC), :], srcs_v)
        pltpu.sync_copy(dst_hbm.at[w, pl.ds(h * HNC, HNC), :], dsts_v)

        def body(i, carry):
            dga = pltpu.async_copy(hp_hbm.at[srcs_v.at[2 * i]], ra, gsa)
            dgb = pltpu.async_copy(hp_hbm.at[srcs_v.at[2 * i + 1]], rb_, gsb)
            dga.wait()
            dsa = pltpu.async_copy(ra, acc.at[dsts_v.at[2 * i]], ssa,
                                   add=True)
            dgb.wait()
            dsb = pltpu.async_copy(rb_, acc.at[dsts_v.at[2 * i + 1]], ssb,
                                   add=True)
            dsa.wait()
            dsb.wait()
            return carry

        lax.fori_loop(0, HNC // 2, body, 0)
    plsc.subcore_barrier()

    def wb(j, carry):
        rbase = s * RPT + j * CH
        pltpu.sync_copy(acc.at[pl.ds(rbase, CH), :], ra)
        pltpu.sync_copy(ra, out_hbm.at[c, pl.ds(rbase, CH), :])
        return carry

    lax.fori_loop(0, RCH, wb, 0)


def _agg_call(hp, srcp, dstp):
    k = pl.kernel(
        _agg_body,
        out_type=jax.ShapeDtypeStruct((NC, NPAD, FD), jnp.float32),
        mesh=_mesh(),
        scratch_types=[
            pltpu.VMEM((HNC, CH), jnp.int32),
            pltpu.VMEM((HNC, CH), jnp.int32),
            pltpu.VMEM((CH, FD), jnp.float32),
            pltpu.VMEM((CH, FD), jnp.float32),
            pltpu.SemaphoreType.DMA,
            pltpu.SemaphoreType.DMA,
            pltpu.SemaphoreType.DMA,
            pltpu.SemaphoreType.DMA,
            pltpu.VMEM_SHARED((NPAD, FD), jnp.float32),
        ],
    )
    return k(hp, srcp, dstp)


def _scale_body(x_ref, w_ref, d0_ref, d1_ref, hp_ref, dm_ref):
    i = pl.program_id(0)
    d0 = d0_ref[pl.ds(i, 1), :]
    d1 = d1_ref[pl.ds(i, 1), :]
    dis_row = lax.rsqrt(d0 + d1 + 1.0)
    ones_row = jnp.ones((1, FD), jnp.float32)
    dm = lax.dot_general(dis_row, ones_row, (((0,), (0,)), ((), ())),
                         preferred_element_type=jnp.float32)
    h = jnp.dot(x_ref[...], w_ref[...], preferred_element_type=jnp.float32)
    dm_ref[...] = dm
    hp_ref[...] = dm * h


def _scale_call(xp, W1, deg0, deg1):
    grid = NPAD // FD
    return pl.pallas_call(
        _scale_body,
        grid=(grid,),
        in_specs=[
            pl.BlockSpec((FD, FD), lambda i: (i, 0)),
            pl.BlockSpec((FD, FD), lambda i: (0, 0)),
            pl.BlockSpec((NPAD // FD, FD), lambda i: (0, 0)),
            pl.BlockSpec((NPAD // FD, FD), lambda i: (0, 0)),
        ],
        out_specs=[
            pl.BlockSpec((FD, FD), lambda i: (i, 0)),
            pl.BlockSpec((FD, FD), lambda i: (i, 0)),
        ],
        out_shape=[
            jax.ShapeDtypeStruct((NPAD, FD), jnp.float32),
            jax.ShapeDtypeStruct((NPAD, FD), jnp.float32),
        ],
    )(xp, W1, deg0, deg1)


def _ln_relu(z, g_ref, be_ref):
    mu = jnp.mean(z, axis=-1, keepdims=True)
    zc = z - mu
    var = jnp.mean(zc * zc, axis=-1, keepdims=True)
    zn = zc * lax.rsqrt(var + EPS) * g_ref[...] + be_ref[...]
    return jnp.maximum(zn, 0.0)


def _mid_body(a0_ref, a1_ref, hp_ref, dm_ref, b_ref, g_ref, be_ref, w2_ref,
              out_ref):
    dm = dm_ref[...]
    z = dm * (a0_ref[...] + a1_ref[...] - hp_ref[...]) + b_ref[...]
    r = _ln_relu(z, g_ref, be_ref)
    h2 = jnp.dot(r, w2_ref[...], preferred_element_type=jnp.float32)
    out_ref[...] = dm * h2


def _mid_call(a0, a1, hp, dm, b1, g1, be1, W2):
    grid = NPAD // FD
    blk = lambda i: (i, 0)
    vec = lambda i: (0, 0)
    return pl.pallas_call(
        _mid_body,
        grid=(grid,),
        in_specs=[
            pl.BlockSpec((FD, FD), blk),
            pl.BlockSpec((FD, FD), blk),
            pl.BlockSpec((FD, FD), blk),
            pl.BlockSpec((FD, FD), blk),
            pl.BlockSpec((1, FD), vec),
            pl.BlockSpec((1, FD), vec),
            pl.BlockSpec((1, FD), vec),
            pl.BlockSpec((FD, FD), vec),
        ],
        out_specs=pl.BlockSpec((FD, FD), blk),
        out_shape=jax.ShapeDtypeStruct((NPAD, FD), jnp.float32),
    )(a0, a1, hp, dm, b1, g1, be1, W2)


def _final_body(a0_ref, a1_ref, hp_ref, dm_ref, b_ref, g_ref, be_ref,
                out_ref):
    i = pl.program_id(0)
    z = dm_ref[...] * (a0_ref[...] + a1_ref[...] - hp_ref[...]) + b_ref[...]
    r = _ln_relu(z, g_ref, be_ref)
    rowid = lax.broadcasted_iota(jnp.int32, (FD, FD), 0) + i * FD
    r = jnp.where(rowid < NN, r, 0.0)
    part = jnp.dot(jnp.ones((1, FD), jnp.float32), r,
                   preferred_element_type=jnp.float32)

    @pl.when(i == 0)
    def _():
        out_ref[...] = jnp.zeros((1, FD), jnp.float32)

    out_ref[...] += part

    @pl.when(i == NPAD // FD - 1)
    def _():
        out_ref[...] = out_ref[...] * (1.0 / NN)


def _final_call(a0, a1, hp, dm, b2, g2, be2):
    grid = NPAD // FD
    blk = lambda i: (i, 0)
    vec = lambda i: (0, 0)
    return pl.pallas_call(
        _final_body,
        grid=(grid,),
        in_specs=[
            pl.BlockSpec((FD, FD), blk),
            pl.BlockSpec((FD, FD), blk),
            pl.BlockSpec((FD, FD), blk),
            pl.BlockSpec((FD, FD), blk),
            pl.BlockSpec((1, FD), vec),
            pl.BlockSpec((1, FD), vec),
            pl.BlockSpec((1, FD), vec),
        ],
        out_specs=pl.BlockSpec((1, FD), vec),
        out_shape=jax.ShapeDtypeStruct((1, FD), jnp.float32),
    )(a0, a1, hp, dm, b2, g2, be2)


def kernel(x, edge_index, W1, b1, g1, be1, W2, b2, g2, be2):
    npad_e = EPAD - NE
    pad_src = (jnp.arange(npad_e, dtype=jnp.int32) * 13) % NN
    pad_dst = NN + (jnp.arange(npad_e, dtype=jnp.int32) % (NPAD - NN))
    srcp = jnp.concatenate([edge_index[0].astype(jnp.int32), pad_src]
                           ).reshape(NW, NCHUNK, CH)
    dstp = jnp.concatenate([edge_index[1].astype(jnp.int32), pad_dst]
                           ).reshape(NW, NCHUNK, CH)
    xp = jnp.pad(x, ((0, NPAD - NN), (0, 0)))

    degs = _deg_call(dstp).reshape(NC, NPAD // FD, FD)
    hp1, dm = _scale_call(xp, W1, degs[0], degs[1])
    acc1 = _agg_call(hp1, srcp, dstp)
    hp2 = _mid_call(acc1[0], acc1[1], hp1, dm,
                    b1.reshape(1, FD), g1.reshape(1, FD), be1.reshape(1, FD),
                    W2)
    acc2 = _agg_call(hp2, srcp, dstp)
    return _final_call(acc2[0], acc2[1], hp2, dm,
                       b2.reshape(1, FD), g2.reshape(1, FD),
                       be2.reshape(1, FD))


# speedup vs baseline: 19.8653x; 1.0402x over previous
import functools

import jax
import jax.numpy as jnp
from jax import lax
from jax.experimental import pallas as pl
from jax.experimental.pallas import tpu as pltpu
from jax.experimental.pallas import tpu_sc as plsc

NN = 10000
FD = 128
NE = 320000
NC = 2
NS = 16
NW = NC * NS
NPAD = 10240
RPT = NPAD // NS
CH = 128
RCH = RPT // CH
EC = 125
ECW = NE // NW // EC
HNC = ECW // 2
EPS = 1e-5


def _mesh():
    return plsc.VectorSubcoreMesh(core_axis_name="c", subcore_axis_name="s")


def _deg_body(dst_hbm, out0_hbm, out1_hbm, dsts_v, ones_v, stg1, stg2,
              sem0, sem1, sem2, sem3, acc):
    c = lax.axis_index("c")
    s = lax.axis_index("s")
    w = c * NS + s
    for t in range(128 // 16):
        ones_v[pl.ds(16 * t, 16)] = jnp.ones((16,), jnp.float32)
    for t in range(RPT // 16):
        stg1[pl.ds(16 * t, 16)] = jnp.zeros((16,), jnp.float32)
    pltpu.sync_copy(stg1, acc.at[pl.ds(s * RPT, RPT)])
    pltpu.sync_copy(dst_hbm.at[pl.ds(w * ECW, ECW), :], dsts_v)
    plsc.subcore_barrier()
    sems = [sem0, sem1, sem2, sem3]

    def body(i, carry):
        descs = []
        for b in range(4):
            descs.append(pltpu.async_copy(
                ones_v.at[pl.ds(0, EC)], acc.at[dsts_v.at[i * 4 + b]],
                sems[b], add=True))
        for d in descs:
            d.wait()
        return carry

    lax.fori_loop(0, ECW // 4, body, 0)
    plsc.subcore_barrier()
    pltpu.sync_copy(acc.at[pl.ds(s * RPT, RPT)], stg1)
    for t in range(RPT // 16):
        stg2[t // 8, pl.ds((t % 8) * 16, 16)] = stg1[pl.ds(16 * t, 16)]

    @pl.when(c == 0)
    def _():
        pltpu.sync_copy(stg2, out0_hbm.at[s])

    @pl.when(c == 1)
    def _():
        pltpu.sync_copy(stg2, out1_hbm.at[s])


def _deg_call(dstp):
    k = pl.kernel(
        _deg_body,
        out_type=(
            jax.ShapeDtypeStruct((NS, RPT // FD, FD), jnp.float32),
            jax.ShapeDtypeStruct((NS, RPT // FD, FD), jnp.float32),
        ),
        mesh=_mesh(),
        scratch_types=[
            pltpu.VMEM((ECW, EC), jnp.int32),
            pltpu.VMEM((128,), jnp.float32),
            pltpu.VMEM((RPT,), jnp.float32),
            pltpu.VMEM((RPT // FD, FD), jnp.float32),
            pltpu.SemaphoreType.DMA,
            pltpu.SemaphoreType.DMA,
            pltpu.SemaphoreType.DMA,
            pltpu.SemaphoreType.DMA,
            pltpu.VMEM_SHARED((NPAD,), jnp.float32),
        ],
    )
    return k(dstp)


def _agg_body(hp_hbm, src_hbm, dst_hbm, out0_hbm, out1_hbm,
              srcs_v, dsts_v, ra, rb_, gsa, gsb, ssa, ssb, acc):
    c = lax.axis_index("c")
    s = lax.axis_index("s")
    w = c * NS + s

    def initb(j, carry):
        rbase = s * RPT + j * CH
        pltpu.sync_copy(hp_hbm.at[pl.ds(rbase, CH), :], ra)
        pltpu.sync_copy(ra, acc.at[pl.ds(rbase, CH), :])
        return carry

    lax.fori_loop(0, RCH, initb, 0)
    plsc.subcore_barrier()

    ras = ra.at[pl.ds(0, EC), :]
    rbs = rb_.at[pl.ds(0, EC), :]
    for h in range(2):
        pltpu.sync_copy(src_hbm.at[pl.ds(w * ECW + h * HNC, HNC), :], srcs_v)
        pltpu.sync_copy(dst_hbm.at[pl.ds(w * ECW + h * HNC, HNC), :], dsts_v)

        def body(i, carry):
            dga = pltpu.async_copy(hp_hbm.at[srcs_v.at[2 * i]], ras, gsa)
            dgb = pltpu.async_copy(hp_hbm.at[srcs_v.at[2 * i + 1]], rbs, gsb)
            dga.wait()
            dsa = pltpu.async_copy(ras, acc.at[dsts_v.at[2 * i]], ssa,
                                   add=True)
            dgb.wait()
            dsb = pltpu.async_copy(rbs, acc.at[dsts_v.at[2 * i + 1]], ssb,
                                   add=True)
            dsa.wait()
            dsb.wait()
            return carry

        lax.fori_loop(0, HNC // 2, body, 0)
    plsc.subcore_barrier()

    def wb0(j, carry):
        rbase = s * RPT + j * CH
        pltpu.sync_copy(acc.at[pl.ds(rbase, CH), :], ra)
        pltpu.sync_copy(ra, out0_hbm.at[pl.ds(rbase, CH), :])
        return carry

    def wb1(j, carry):
        rbase = s * RPT + j * CH
        pltpu.sync_copy(acc.at[pl.ds(rbase, CH), :], ra)
        pltpu.sync_copy(ra, out1_hbm.at[pl.ds(rbase, CH), :])
        return carry

    @pl.when(c == 0)
    def _():
        lax.fori_loop(0, RCH, wb0, 0)

    @pl.when(c == 1)
    def _():
        lax.fori_loop(0, RCH, wb1, 0)


def _agg_call(hp, srcp, dstp):
    k = pl.kernel(
        _agg_body,
        out_type=(
            jax.ShapeDtypeStruct((NPAD, FD), jnp.float32),
            jax.ShapeDtypeStruct((NPAD, FD), jnp.float32),
        ),
        mesh=_mesh(),
        scratch_types=[
            pltpu.VMEM((HNC, EC), jnp.int32),
            pltpu.VMEM((HNC, EC), jnp.int32),
            pltpu.VMEM((CH, FD), jnp.float32),
            pltpu.VMEM((CH, FD), jnp.float32),
            pltpu.SemaphoreType.DMA,
            pltpu.SemaphoreType.DMA,
            pltpu.SemaphoreType.DMA,
            pltpu.SemaphoreType.DMA,
            pltpu.VMEM_SHARED((NPAD, FD), jnp.float32),
        ],
    )
    return k(hp, srcp, dstp)


def _scale_body(x_ref, w_ref, d0_ref, d1_ref, hp_ref, dm_ref):
    i = pl.program_id(0)
    d0 = d0_ref[pl.ds(i, 1), :]
    d1 = d1_ref[pl.ds(i, 1), :]
    dis_row = lax.rsqrt(d0 + d1 + 1.0)
    ones_row = jnp.ones((1, FD), jnp.float32)
    dm = lax.dot_general(dis_row, ones_row, (((0,), (0,)), ((), ())),
                         preferred_element_type=jnp.float32)
    h = jnp.dot(x_ref[...], w_ref[...], preferred_element_type=jnp.float32)
    dm_ref[...] = dm
    hp_ref[...] = dm * h


def _scale_call(xp, W1, deg0, deg1):
    grid = NPAD // FD
    return pl.pallas_call(
        _scale_body,
        grid=(grid,),
        in_specs=[
            pl.BlockSpec((FD, FD), lambda i: (i, 0)),
            pl.BlockSpec((FD, FD), lambda i: (0, 0)),
            pl.BlockSpec((NPAD // FD, FD), lambda i: (0, 0)),
            pl.BlockSpec((NPAD // FD, FD), lambda i: (0, 0)),
        ],
        out_specs=[
            pl.BlockSpec((FD, FD), lambda i: (i, 0)),
            pl.BlockSpec((FD, FD), lambda i: (i, 0)),
        ],
        out_shape=[
            jax.ShapeDtypeStruct((NPAD, FD), jnp.float32),
            jax.ShapeDtypeStruct((NPAD, FD), jnp.float32),
        ],
    )(xp, W1, deg0, deg1)


def _ln_relu(z, g_ref, be_ref):
    mu = jnp.mean(z, axis=-1, keepdims=True)
    zc = z - mu
    var = jnp.mean(zc * zc, axis=-1, keepdims=True)
    zn = zc * lax.rsqrt(var + EPS) * g_ref[...] + be_ref[...]
    return jnp.maximum(zn, 0.0)


def _mid_body(a0_ref, a1_ref, hp_ref, dm_ref, b_ref, g_ref, be_ref, w2_ref,
              out_ref):
    dm = dm_ref[...]
    z = dm * (a0_ref[...] + a1_ref[...] - hp_ref[...]) + b_ref[...]
    r = _ln_relu(z, g_ref, be_ref)
    h2 = jnp.dot(r, w2_ref[...], preferred_element_type=jnp.float32)
    out_ref[...] = dm * h2


def _mid_call(a0, a1, hp, dm, b1, g1, be1, W2):
    grid = NPAD // FD
    blk = lambda i: (i, 0)
    vec = lambda i: (0, 0)
    return pl.pallas_call(
        _mid_body,
        grid=(grid,),
        in_specs=[
            pl.BlockSpec((FD, FD), blk),
            pl.BlockSpec((FD, FD), blk),
            pl.BlockSpec((FD, FD), blk),
            pl.BlockSpec((FD, FD), blk),
            pl.BlockSpec((1, FD), vec),
            pl.BlockSpec((1, FD), vec),
            pl.BlockSpec((1, FD), vec),
            pl.BlockSpec((FD, FD), vec),
        ],
        out_specs=pl.BlockSpec((FD, FD), blk),
        out_shape=jax.ShapeDtypeStruct((NPAD, FD), jnp.float32),
    )(a0, a1, hp, dm, b1, g1, be1, W2)


def _final_body(a0_ref, a1_ref, hp_ref, dm_ref, b_ref, g_ref, be_ref,
                out_ref):
    i = pl.program_id(0)
    z = dm_ref[...] * (a0_ref[...] + a1_ref[...] - hp_ref[...]) + b_ref[...]
    r = _ln_relu(z, g_ref, be_ref)
    rowid = lax.broadcasted_iota(jnp.int32, (FD, FD), 0) + i * FD
    r = jnp.where(rowid < NN, r, 0.0)
    part = jnp.dot(jnp.ones((1, FD), jnp.float32), r,
                   preferred_element_type=jnp.float32)

    @pl.when(i == 0)
    def _():
        out_ref[...] = jnp.zeros((1, FD), jnp.float32)

    out_ref[...] += part

    @pl.when(i == NPAD // FD - 1)
    def _():
        out_ref[...] = out_ref[...] * (1.0 / NN)


def _final_call(a0, a1, hp, dm, b2, g2, be2):
    grid = NPAD // FD
    blk = lambda i: (i, 0)
    vec = lambda i: (0, 0)
    return pl.pallas_call(
        _final_body,
        grid=(grid,),
        in_specs=[
            pl.BlockSpec((FD, FD), blk),
            pl.BlockSpec((FD, FD), blk),
            pl.BlockSpec((FD, FD), blk),
            pl.BlockSpec((FD, FD), blk),
            pl.BlockSpec((1, FD), vec),
            pl.BlockSpec((1, FD), vec),
            pl.BlockSpec((1, FD), vec),
        ],
        out_specs=pl.BlockSpec((1, FD), vec),
        out_shape=jax.ShapeDtypeStruct((1, FD), jnp.float32),
    )(a0, a1, hp, dm, b2, g2, be2)


def kernel(x, edge_index, W1, b1, g1, be1, W2, b2, g2, be2):
    srcp = edge_index[0].astype(jnp.int32).reshape(NW * ECW, EC)
    dstp = edge_index[1].astype(jnp.int32).reshape(NW * ECW, EC)
    xp = jnp.pad(x, ((0, NPAD - NN), (0, 0)))

    deg0, deg1 = _deg_call(dstp)
    hp1, dm = _scale_call(xp, W1,
                          deg0.reshape(NPAD // FD, FD),
                          deg1.reshape(NPAD // FD, FD))
    a10, a11 = _agg_call(hp1, srcp, dstp)
    hp2 = _mid_call(a10, a11, hp1, dm,
                    b1.reshape(1, FD), g1.reshape(1, FD), be1.reshape(1, FD),
                    W2)
    a20, a21 = _agg_call(hp2, srcp, dstp)
    return _final_call(a20, a21, hp2, dm,
                       b2.reshape(1, FD), g2.reshape(1, FD),
                       be2.reshape(1, FD))


# speedup vs baseline: 21.7388x; 1.0943x over previous
import functools

import jax
import jax.numpy as jnp
from jax import lax
from jax.experimental import pallas as pl
from jax.experimental.pallas import tpu as pltpu
from jax.experimental.pallas import tpu_sc as plsc

NN = 10000
FD = 128
NE = 320000
NC = 2
NS = 16
NW = NC * NS
NPAD = 10240
RPT = NPAD // NS
CH = 128
RCH = RPT // CH
EC = 125
ECW = NE // NW // EC
HNC = ECW // 2
EPS = 1e-5


def _mesh():
    return plsc.VectorSubcoreMesh(core_axis_name="c", subcore_axis_name="s")


def _deg_body(dst_hbm, out0_hbm, out1_hbm, dsts_v, ones_v, stg1, stg2,
              sem0, sem1, sem2, sem3, acc):
    c = lax.axis_index("c")
    s = lax.axis_index("s")
    w = c * NS + s
    for t in range(128 // 16):
        ones_v[pl.ds(16 * t, 16)] = jnp.ones((16,), jnp.float32)
    for t in range(RPT // 16):
        stg1[pl.ds(16 * t, 16)] = jnp.zeros((16,), jnp.float32)
    pltpu.sync_copy(stg1, acc.at[pl.ds(s * RPT, RPT)])
    pltpu.sync_copy(dst_hbm.at[pl.ds(w * ECW, ECW), :], dsts_v)
    plsc.subcore_barrier()
    sems = [sem0, sem1, sem2, sem3]

    def body(i, carry):
        descs = []
        for b in range(4):
            descs.append(pltpu.async_copy(
                ones_v.at[pl.ds(0, EC)], acc.at[dsts_v.at[i * 4 + b]],
                sems[b], add=True))
        for d in descs:
            d.wait()
        return carry

    lax.fori_loop(0, ECW // 4, body, 0)
    plsc.subcore_barrier()
    pltpu.sync_copy(acc.at[pl.ds(s * RPT, RPT)], stg1)
    for t in range(RPT // 16):
        stg2[t // 8, pl.ds((t % 8) * 16, 16)] = stg1[pl.ds(16 * t, 16)]

    @pl.when(c == 0)
    def _():
        pltpu.sync_copy(stg2, out0_hbm.at[s])

    @pl.when(c == 1)
    def _():
        pltpu.sync_copy(stg2, out1_hbm.at[s])


def _deg_call(dstp):
    k = pl.kernel(
        _deg_body,
        out_type=(
            jax.ShapeDtypeStruct((NS, RPT // FD, FD), jnp.float32),
            jax.ShapeDtypeStruct((NS, RPT // FD, FD), jnp.float32),
        ),
        mesh=_mesh(),
        scratch_types=[
            pltpu.VMEM((ECW, EC), jnp.int32),
            pltpu.VMEM((128,), jnp.float32),
            pltpu.VMEM((RPT,), jnp.float32),
            pltpu.VMEM((RPT // FD, FD), jnp.float32),
            pltpu.SemaphoreType.DMA,
            pltpu.SemaphoreType.DMA,
            pltpu.SemaphoreType.DMA,
            pltpu.SemaphoreType.DMA,
            pltpu.VMEM_SHARED((NPAD,), jnp.float32),
        ],
    )
    return k(dstp)


def _agg_body(hp_hbm, src_hbm, dst_hbm, out0_hbm, out1_hbm,
              srcs_v, dsts_v, ra, rb_, gsa, gsb, ssa, ssb, acc):
    c = lax.axis_index("c")
    s = lax.axis_index("s")
    w = c * NS + s

    def _ld(j, buf, sem):
        return pltpu.async_copy(hp_hbm.at[pl.ds(s * RPT + j * CH, CH), :],
                                buf, sem)

    dl = {0: _ld(0, ra, gsa), 1: _ld(1, rb_, gsb)}
    for j in range(RCH):
        buf, gsem, ssem = (ra, gsa, ssa) if j % 2 == 0 else (rb_, gsb, ssb)
        dl[j].wait()
        pltpu.async_copy(buf, acc.at[pl.ds(s * RPT + j * CH, CH), :],
                         ssem).wait()
        if j + 2 < RCH:
            dl[j + 2] = _ld(j + 2, buf, gsem)
    plsc.subcore_barrier()

    ras = ra.at[pl.ds(0, EC), :]
    rbs = rb_.at[pl.ds(0, EC), :]
    for h in range(2):
        pltpu.sync_copy(src_hbm.at[pl.ds(w * ECW + h * HNC, HNC), :], srcs_v)
        pltpu.sync_copy(dst_hbm.at[pl.ds(w * ECW + h * HNC, HNC), :], dsts_v)
        pltpu.async_copy(hp_hbm.at[srcs_v.at[0]], ras, gsa)

        def body(i, carry):
            @pl.when(i > 0)
            def _():
                pltpu.make_async_copy(
                    rbs, acc.at[dsts_v.at[2 * i - 1]], ssb).wait()

            pltpu.make_async_copy(
                hp_hbm.at[srcs_v.at[2 * i]], ras, gsa).wait()
            pltpu.async_copy(ras, acc.at[dsts_v.at[2 * i]], ssa, add=True)
            dgb = pltpu.async_copy(hp_hbm.at[srcs_v.at[2 * i + 1]], rbs, gsb)
            dgb.wait()
            pltpu.make_async_copy(ras, acc.at[dsts_v.at[2 * i]], ssa).wait()
            pltpu.async_copy(rbs, acc.at[dsts_v.at[2 * i + 1]], ssb,
                             add=True)

            @pl.when(i < HNC // 2 - 1)
            def _():
                pltpu.async_copy(hp_hbm.at[srcs_v.at[2 * i + 2]], ras, gsa)

            return carry

        lax.fori_loop(0, HNC // 2, body, 0)
        pltpu.make_async_copy(rbs, acc.at[dsts_v.at[HNC - 1]], ssb).wait()
    plsc.subcore_barrier()

    def _wb(out_hbm):
        def _ld2(j, buf, sem):
            return pltpu.async_copy(acc.at[pl.ds(s * RPT + j * CH, CH), :],
                                    buf, sem)

        dl2 = {0: _ld2(0, ra, gsa), 1: _ld2(1, rb_, gsb)}
        for j in range(RCH):
            buf, gsem, ssem = ((ra, gsa, ssa) if j % 2 == 0
                               else (rb_, gsb, ssb))
            dl2[j].wait()
            pltpu.async_copy(buf, out_hbm.at[pl.ds(s * RPT + j * CH, CH), :],
                             ssem).wait()
            if j + 2 < RCH:
                dl2[j + 2] = _ld2(j + 2, buf, gsem)

    @pl.when(c == 0)
    def _():
        _wb(out0_hbm)

    @pl.when(c == 1)
    def _():
        _wb(out1_hbm)


def _agg_call(hp, srcp, dstp):
    k = pl.kernel(
        _agg_body,
        out_type=(
            jax.ShapeDtypeStruct((NPAD, FD), jnp.float32),
            jax.ShapeDtypeStruct((NPAD, FD), jnp.float32),
        ),
        mesh=_mesh(),
        scratch_types=[
            pltpu.VMEM((HNC, EC), jnp.int32),
            pltpu.VMEM((HNC, EC), jnp.int32),
            pltpu.VMEM((CH, FD), jnp.float32),
            pltpu.VMEM((CH, FD), jnp.float32),
            pltpu.SemaphoreType.DMA,
            pltpu.SemaphoreType.DMA,
            pltpu.SemaphoreType.DMA,
            pltpu.SemaphoreType.DMA,
            pltpu.VMEM_SHARED((NPAD, FD), jnp.float32),
        ],
    )
    return k(hp, srcp, dstp)


def _scale_body(x_ref, w_ref, d0_ref, d1_ref, hp_ref, dm_ref):
    i = pl.program_id(0)
    d0 = d0_ref[pl.ds(i, 1), :]
    d1 = d1_ref[pl.ds(i, 1), :]
    dis_row = lax.rsqrt(d0 + d1 + 1.0)
    ones_row = jnp.ones((1, FD), jnp.float32)
    dm = lax.dot_general(dis_row, ones_row, (((0,), (0,)), ((), ())),
                         preferred_element_type=jnp.float32)
    h = jnp.dot(x_ref[...], w_ref[...], preferred_element_type=jnp.float32)
    dm_ref[...] = dm
    hp_ref[...] = dm * h


def _scale_call(xp, W1, deg0, deg1):
    grid = NPAD // FD
    return pl.pallas_call(
        _scale_body,
        grid=(grid,),
        in_specs=[
            pl.BlockSpec((FD, FD), lambda i: (i, 0)),
            pl.BlockSpec((FD, FD), lambda i: (0, 0)),
            pl.BlockSpec((NPAD // FD, FD), lambda i: (0, 0)),
            pl.BlockSpec((NPAD // FD, FD), lambda i: (0, 0)),
        ],
        out_specs=[
            pl.BlockSpec((FD, FD), lambda i: (i, 0)),
            pl.BlockSpec((FD, FD), lambda i: (i, 0)),
        ],
        out_shape=[
            jax.ShapeDtypeStruct((NPAD, FD), jnp.float32),
            jax.ShapeDtypeStruct((NPAD, FD), jnp.float32),
        ],
    )(xp, W1, deg0, deg1)


def _ln_relu(z, g_ref, be_ref):
    mu = jnp.mean(z, axis=-1, keepdims=True)
    zc = z - mu
    var = jnp.mean(zc * zc, axis=-1, keepdims=True)
    zn = zc * lax.rsqrt(var + EPS) * g_ref[...] + be_ref[...]
    return jnp.maximum(zn, 0.0)


def _mid_body(a0_ref, a1_ref, hp_ref, dm_ref, b_ref, g_ref, be_ref, w2_ref,
              out_ref):
    dm = dm_ref[...]
    z = dm * (a0_ref[...] + a1_ref[...] - hp_ref[...]) + b_ref[...]
    r = _ln_relu(z, g_ref, be_ref)
    h2 = jnp.dot(r, w2_ref[...], preferred_element_type=jnp.float32)
    out_ref[...] = dm * h2


def _mid_call(a0, a1, hp, dm, b1, g1, be1, W2):
    grid = NPAD // FD
    blk = lambda i: (i, 0)
    vec = lambda i: (0, 0)
    return pl.pallas_call(
        _mid_body,
        grid=(grid,),
        in_specs=[
            pl.BlockSpec((FD, FD), blk),
            pl.BlockSpec((FD, FD), blk),
            pl.BlockSpec((FD, FD), blk),
            pl.BlockSpec((FD, FD), blk),
            pl.BlockSpec((1, FD), vec),
            pl.BlockSpec((1, FD), vec),
            pl.BlockSpec((1, FD), vec),
            pl.BlockSpec((FD, FD), vec),
        ],
        out_specs=pl.BlockSpec((FD, FD), blk),
        out_shape=jax.ShapeDtypeStruct((NPAD, FD), jnp.float32),
    )(a0, a1, hp, dm, b1, g1, be1, W2)


def _final_body(a0_ref, a1_ref, hp_ref, dm_ref, b_ref, g_ref, be_ref,
                out_ref):
    i = pl.program_id(0)
    z = dm_ref[...] * (a0_ref[...] + a1_ref[...] - hp_ref[...]) + b_ref[...]
    r = _ln_relu(z, g_ref, be_ref)
    rowid = lax.broadcasted_iota(jnp.int32, (FD, FD), 0) + i * FD
    r = jnp.where(rowid < NN, r, 0.0)
    part = jnp.dot(jnp.ones((1, FD), jnp.float32), r,
                   preferred_element_type=jnp.float32)

    @pl.when(i == 0)
    def _():
        out_ref[...] = jnp.zeros((1, FD), jnp.float32)

    out_ref[...] += part

    @pl.when(i == NPAD // FD - 1)
    def _():
        out_ref[...] = out_ref[...] * (1.0 / NN)


def _final_call(a0, a1, hp, dm, b2, g2, be2):
    grid = NPAD // FD
    blk = lambda i: (i, 0)
    vec = lambda i: (0, 0)
    return pl.pallas_call(
        _final_body,
        grid=(grid,),
        in_specs=[
            pl.BlockSpec((FD, FD), blk),
            pl.BlockSpec((FD, FD), blk),
            pl.BlockSpec((FD, FD), blk),
            pl.BlockSpec((FD, FD), blk),
            pl.BlockSpec((1, FD), vec),
            pl.BlockSpec((1, FD), vec),
            pl.BlockSpec((1, FD), vec),
        ],
        out_specs=pl.BlockSpec((1, FD), vec),
        out_shape=jax.ShapeDtypeStruct((1, FD), jnp.float32),
    )(a0, a1, hp, dm, b2, g2, be2)


def kernel(x, edge_index, W1, b1, g1, be1, W2, b2, g2, be2):
    srcp = edge_index[0].astype(jnp.int32).reshape(NW * ECW, EC)
    dstp = edge_index[1].astype(jnp.int32).reshape(NW * ECW, EC)
    xp = jnp.pad(x, ((0, NPAD - NN), (0, 0)))

    deg0, deg1 = _deg_call(dstp)
    hp1, dm = _scale_call(xp, W1,
                          deg0.reshape(NPAD // FD, FD),
                          deg1.reshape(NPAD // FD, FD))
    a10, a11 = _agg_call(hp1, srcp, dstp)
    hp2 = _mid_call(a10, a11, hp1, dm,
                    b1.reshape(1, FD), g1.reshape(1, FD), be1.reshape(1, FD),
                    W2)
    a20, a21 = _agg_call(hp2, srcp, dstp)
    return _final_call(a20, a21, hp2, dm,
                       b2.reshape(1, FD), g2.reshape(1, FD),
                       be2.reshape(1, FD))


# speedup vs baseline: 27.1896x; 1.2507x over previous
import functools

import jax
import jax.numpy as jnp
from jax import lax
from jax.experimental import pallas as pl
from jax.experimental.pallas import tpu as pltpu
from jax.experimental.pallas import tpu_sc as plsc

NN = 10000
FD = 128
NE = 320000
NC = 2
NS = 16
NW = NC * NS
NPAD = 10240
RPT = NPAD // NS
CH = 128
RCH = RPT // CH
EC = 125
ECW = NE // NW // EC
HNC = ECW // 2
EPS = 1e-5


def _mesh():
    return plsc.VectorSubcoreMesh(core_axis_name="c", subcore_axis_name="s")


def _deg_body(dst_hbm, out0_hbm, out1_hbm, dsts_v, ones_v, stg1,
              sem0, sem1, sem2, sem3, acc):
    c = lax.axis_index("c")
    s = lax.axis_index("s")
    w = c * NS + s
    slab = pl.ds(s * RPT, RPT)
    for t in range(128 // 16):
        ones_v[pl.ds(16 * t, 16)] = jnp.ones((16,), jnp.float32)
    for t in range(RPT // 16):
        stg1[pl.ds(16 * t, 16)] = jnp.zeros((16,), jnp.float32)
    pltpu.sync_copy(stg1, acc.at[slab])
    pltpu.sync_copy(dst_hbm.at[pl.ds(w * ECW, ECW), :], dsts_v)
    plsc.subcore_barrier()
    sems = [sem0, sem1, sem2, sem3]

    def body(i, carry):
        descs = []
        for b in range(4):
            descs.append(pltpu.async_copy(
                ones_v.at[pl.ds(0, EC)], acc.at[dsts_v.at[i * 4 + b]],
                sems[b], add=True))
        for d in descs:
            d.wait()
        return carry

    lax.fori_loop(0, ECW // 4, body, 0)
    plsc.subcore_barrier()
    pltpu.sync_copy(acc.at[slab], stg1)

    @pl.when(c == 0)
    def _():
        pltpu.sync_copy(stg1, out0_hbm.at[slab])

    @pl.when(c == 1)
    def _():
        pltpu.sync_copy(stg1, out1_hbm.at[slab])


def _deg_call(dstp):
    k = pl.kernel(
        _deg_body,
        out_type=(
            jax.ShapeDtypeStruct((NPAD,), jnp.float32),
            jax.ShapeDtypeStruct((NPAD,), jnp.float32),
        ),
        mesh=_mesh(),
        scratch_types=[
            pltpu.VMEM((ECW, EC), jnp.int32),
            pltpu.VMEM((128,), jnp.float32),
            pltpu.VMEM((RPT,), jnp.float32),
            pltpu.SemaphoreType.DMA,
            pltpu.SemaphoreType.DMA,
            pltpu.SemaphoreType.DMA,
            pltpu.SemaphoreType.DMA,
            pltpu.VMEM_SHARED((NPAD,), jnp.float32),
        ],
    )
    return k(dstp)


def _agg_body(hp_hbm, src_hbm, dst_hbm, out0_hbm, out1_hbm,
              srcs_v, dsts_v, ra, rb_, gsa, gsb, ssa, ssb, acc):
    c = lax.axis_index("c")
    s = lax.axis_index("s")
    w = c * NS + s

    def _ld(j, buf, sem):
        return pltpu.async_copy(hp_hbm.at[pl.ds(s * RPT + j * CH, CH), :],
                                buf, sem)

    dl = {0: _ld(0, ra, gsa), 1: _ld(1, rb_, gsb)}
    for j in range(RCH):
        buf, gsem, ssem = (ra, gsa, ssa) if j % 2 == 0 else (rb_, gsb, ssb)
        dl[j].wait()
        pltpu.async_copy(buf, acc.at[pl.ds(s * RPT + j * CH, CH), :],
                         ssem).wait()
        if j + 2 < RCH:
            dl[j + 2] = _ld(j + 2, buf, gsem)
    plsc.subcore_barrier()

    ras = ra.at[pl.ds(0, EC), :]
    rbs = rb_.at[pl.ds(0, EC), :]
    for h in range(2):
        pltpu.sync_copy(src_hbm.at[pl.ds(w * ECW + h * HNC, HNC), :], srcs_v)
        pltpu.sync_copy(dst_hbm.at[pl.ds(w * ECW + h * HNC, HNC), :], dsts_v)
        pltpu.async_copy(hp_hbm.at[srcs_v.at[0]], ras, gsa)

        def body(i, carry):
            @pl.when(i > 0)
            def _():
                pltpu.make_async_copy(
                    rbs, acc.at[dsts_v.at[2 * i - 1]], ssb).wait()

            pltpu.make_async_copy(
                hp_hbm.at[srcs_v.at[2 * i]], ras, gsa).wait()
            pltpu.async_copy(ras, acc.at[dsts_v.at[2 * i]], ssa, add=True)
            dgb = pltpu.async_copy(hp_hbm.at[srcs_v.at[2 * i + 1]], rbs, gsb)
            dgb.wait()
            pltpu.make_async_copy(ras, acc.at[dsts_v.at[2 * i]], ssa).wait()
            pltpu.async_copy(rbs, acc.at[dsts_v.at[2 * i + 1]], ssb,
                             add=True)

            @pl.when(i < HNC // 2 - 1)
            def _():
                pltpu.async_copy(hp_hbm.at[srcs_v.at[2 * i + 2]], ras, gsa)

            return carry

        lax.fori_loop(0, HNC // 2, body, 0)
        pltpu.make_async_copy(rbs, acc.at[dsts_v.at[HNC - 1]], ssb).wait()
    plsc.subcore_barrier()

    def _wb(out_hbm):
        def _ld2(j, buf, sem):
            return pltpu.async_copy(acc.at[pl.ds(s * RPT + j * CH, CH), :],
                                    buf, sem)

        dl2 = {0: _ld2(0, ra, gsa), 1: _ld2(1, rb_, gsb)}
        for j in range(RCH):
            buf, gsem, ssem = ((ra, gsa, ssa) if j % 2 == 0
                               else (rb_, gsb, ssb))
            dl2[j].wait()
            pltpu.async_copy(buf, out_hbm.at[pl.ds(s * RPT + j * CH, CH), :],
                             ssem).wait()
            if j + 2 < RCH:
                dl2[j + 2] = _ld2(j + 2, buf, gsem)

    @pl.when(c == 0)
    def _():
        _wb(out0_hbm)

    @pl.when(c == 1)
    def _():
        _wb(out1_hbm)


def _agg_call(hp, srcp, dstp):
    k = pl.kernel(
        _agg_body,
        out_type=(
            jax.ShapeDtypeStruct((NPAD, FD), jnp.float32),
            jax.ShapeDtypeStruct((NPAD, FD), jnp.float32),
        ),
        mesh=_mesh(),
        scratch_types=[
            pltpu.VMEM((HNC, EC), jnp.int32),
            pltpu.VMEM((HNC, EC), jnp.int32),
            pltpu.VMEM((CH, FD), jnp.float32),
            pltpu.VMEM((CH, FD), jnp.float32),
            pltpu.SemaphoreType.DMA,
            pltpu.SemaphoreType.DMA,
            pltpu.SemaphoreType.DMA,
            pltpu.SemaphoreType.DMA,
            pltpu.VMEM_SHARED((NPAD, FD), jnp.float32),
        ],
    )
    return k(hp, srcp, dstp)


BR = 512
TGRID = NPAD // BR


def _dis(d0_ref, d1_ref):
    return lax.rsqrt(d0_ref[...] + d1_ref[...] + 1.0)


def _scale_body(x_ref, w_ref, d0_ref, d1_ref, hp_ref):
    h = jnp.dot(x_ref[...], w_ref[...], preferred_element_type=jnp.float32)
    hp_ref[...] = _dis(d0_ref, d1_ref) * h


def _scale_call(x, W1, deg0, deg1):
    blk = lambda i: (i, 0)
    return pl.pallas_call(
        _scale_body,
        grid=(TGRID,),
        in_specs=[
            pl.BlockSpec((BR, FD), blk),
            pl.BlockSpec((FD, FD), lambda i: (0, 0)),
            pl.BlockSpec((BR, 1), blk),
            pl.BlockSpec((BR, 1), blk),
        ],
        out_specs=pl.BlockSpec((BR, FD), blk),
        out_shape=jax.ShapeDtypeStruct((NPAD, FD), jnp.float32),
    )(x, W1, deg0, deg1)


def _ln_relu(z, g_ref, be_ref):
    ones_col = jnp.full((FD, 1), 1.0 / FD, jnp.float32)
    mu = jnp.dot(z, ones_col, preferred_element_type=jnp.float32)
    s2 = jnp.dot(z * z, ones_col, preferred_element_type=jnp.float32)
    rs = lax.rsqrt(s2 - mu * mu + EPS)
    g = g_ref[...]
    amat = jnp.dot(rs, g, preferred_element_type=jnp.float32)
    cmat = jnp.dot(mu * rs, g, preferred_element_type=jnp.float32) - be_ref[...]
    return jnp.maximum(z * amat - cmat, 0.0)


def _mid_body(a0_ref, a1_ref, hp_ref, d0_ref, d1_ref, b_ref, g_ref, be_ref,
              w2_ref, out_ref):
    dis = _dis(d0_ref, d1_ref)
    z = dis * (a0_ref[...] + a1_ref[...] - hp_ref[...]) + b_ref[...]
    r = _ln_relu(z, g_ref, be_ref)
    h2 = jnp.dot(r, w2_ref[...], preferred_element_type=jnp.float32)
    out_ref[...] = dis * h2


def _mid_call(a0, a1, hp, deg0, deg1, b1, g1, be1, W2):
    blk = lambda i: (i, 0)
    vec = lambda i: (0, 0)
    return pl.pallas_call(
        _mid_body,
        grid=(TGRID,),
        in_specs=[
            pl.BlockSpec((BR, FD), blk),
            pl.BlockSpec((BR, FD), blk),
            pl.BlockSpec((BR, FD), blk),
            pl.BlockSpec((BR, 1), blk),
            pl.BlockSpec((BR, 1), blk),
            pl.BlockSpec((1, FD), vec),
            pl.BlockSpec((1, FD), vec),
            pl.BlockSpec((1, FD), vec),
            pl.BlockSpec((FD, FD), vec),
        ],
        out_specs=pl.BlockSpec((BR, FD), blk),
        out_shape=jax.ShapeDtypeStruct((NPAD, FD), jnp.float32),
    )(a0, a1, hp, deg0, deg1, b1, g1, be1, W2)


def _final_body(a0_ref, a1_ref, hp_ref, d0_ref, d1_ref, b_ref, g_ref, be_ref,
                out_ref):
    i = pl.program_id(0)
    dis = _dis(d0_ref, d1_ref)
    z = dis * (a0_ref[...] + a1_ref[...] - hp_ref[...]) + b_ref[...]
    r = _ln_relu(z, g_ref, be_ref)
    rowid = lax.broadcasted_iota(jnp.int32, (BR, FD), 0) + i * BR
    r = jnp.where(rowid < NN, r, 0.0)
    part = jnp.dot(jnp.ones((1, BR), jnp.float32), r,
                   preferred_element_type=jnp.float32)

    @pl.when(i == 0)
    def _():
        out_ref[...] = jnp.zeros((1, FD), jnp.float32)

    out_ref[...] += part

    @pl.when(i == TGRID - 1)
    def _():
        out_ref[...] = out_ref[...] * (1.0 / NN)


def _final_call(a0, a1, hp, deg0, deg1, b2, g2, be2):
    blk = lambda i: (i, 0)
    vec = lambda i: (0, 0)
    return pl.pallas_call(
        _final_body,
        grid=(TGRID,),
        in_specs=[
            pl.BlockSpec((BR, FD), blk),
            pl.BlockSpec((BR, FD), blk),
            pl.BlockSpec((BR, FD), blk),
            pl.BlockSpec((BR, 1), blk),
            pl.BlockSpec((BR, 1), blk),
            pl.BlockSpec((1, FD), vec),
            pl.BlockSpec((1, FD), vec),
            pl.BlockSpec((1, FD), vec),
        ],
        out_specs=pl.BlockSpec((1, FD), vec),
        out_shape=jax.ShapeDtypeStruct((1, FD), jnp.float32),
    )(a0, a1, hp, deg0, deg1, b2, g2, be2)


def kernel(x, edge_index, W1, b1, g1, be1, W2, b2, g2, be2):
    srcp = edge_index[0].astype(jnp.int32).reshape(NW * ECW, EC)
    dstp = edge_index[1].astype(jnp.int32).reshape(NW * ECW, EC)

    deg0, deg1 = _deg_call(dstp)
    deg0 = deg0.reshape(NPAD, 1)
    deg1 = deg1.reshape(NPAD, 1)
    hp1 = _scale_call(x, W1, deg0, deg1)
    a10, a11 = _agg_call(hp1, srcp, dstp)
    hp2 = _mid_call(a10, a11, hp1, deg0, deg1,
                    b1.reshape(1, FD), g1.reshape(1, FD), be1.reshape(1, FD),
                    W2)
    a20, a21 = _agg_call(hp2, srcp, dstp)
    return _final_call(a20, a21, hp2, deg0, deg1,
                       b2.reshape(1, FD), g2.reshape(1, FD),
                       be2.reshape(1, FD))
